# baseline (speedup 1.0000x reference)
import jax
import jax.numpy as jnp
from jax.experimental import pallas as pl
from jax.experimental.pallas import tpu as pltpu

_B, _L, _ZD = 16, 196, 256
_K = 1024
_H = 1024
_OD = 768
_LT = 49
_R = _LT * _B
_GRID = _L // _LT


def _vq_body(x_ref, embd_ref, w1_ref, b1_ref, w2_ref, b2_ref,
             out_ref, zemb_ref,
             et_ref, e012_ref, t2h_ref, w1s_ref, w2s_ref):
    @pl.when(pl.program_id(0) == 0)
    def _prep():
        embd = embd_ref[...]
        et = embd.T
        et_ref[...] = et.astype(jnp.bfloat16)
        t2h_ref[...] = 0.5 * jnp.sum(et * et, axis=0, keepdims=True)
        e0 = embd.astype(jnp.bfloat16)
        r1 = embd - e0.astype(jnp.float32)
        e1 = r1.astype(jnp.bfloat16)
        r2 = r1 - e1.astype(jnp.float32)
        e012_ref[...] = jnp.concatenate(
            [e0, e1, r2.astype(jnp.bfloat16)], axis=1)
        w1s_ref[...] = w1_ref[...].astype(jnp.bfloat16)
        w2s_ref[...] = w2_ref[...].astype(jnp.bfloat16)

    dims = (((1,), (0,)), ((), ()))
    x2 = x_ref[...].reshape(_R, _ZD)
    _C = _R // 2
    outs, zembs = [], []
    for j in range(2):
        x = x2[j * _C:(j + 1) * _C]
        cross = jax.lax.dot_general(
            x.astype(jnp.bfloat16), et_ref[...], dims,
            preferred_element_type=jnp.float32)
        score = t2h_ref[...] - cross
        idx = jnp.argmin(score, axis=1)
        iota = jax.lax.broadcasted_iota(jnp.int32, (_C, _K), 1)
        onehot = (iota == idx[:, None]).astype(jnp.bfloat16)
        g = jax.lax.dot_general(onehot, e012_ref[...], dims,
                                preferred_element_type=jnp.float32)
        z_emb = g[:, :_ZD] + g[:, _ZD:2 * _ZD] + g[:, 2 * _ZD:]
        zembs.append(z_emb)
        h = jax.lax.dot_general(
            z_emb.astype(jnp.bfloat16), w1s_ref[...], dims,
            preferred_element_type=jnp.float32) + b1_ref[...]
        h = jnp.maximum(h, 0.1 * h)
        outs.append(jax.lax.dot_general(
            h.astype(jnp.bfloat16), w2s_ref[...], dims,
            preferred_element_type=jnp.float32) + b2_ref[...])
    zemb_ref[...] = jnp.concatenate(zembs, 0).reshape(_LT, _B, _ZD)
    out_ref[...] = jnp.concatenate(outs, 0).reshape(_LT, _B, _OD)


@jax.jit
def kernel(X, embd, W1, b1, W2, b2):
    xt = jnp.transpose(X, (1, 0, 2))
    out_t, zemb_t = pl.pallas_call(
        _vq_body,
        grid=(_GRID,),
        in_specs=[
            pl.BlockSpec((_LT, _B, _ZD), lambda i: (i, 0, 0)),
            pl.BlockSpec((_K, _ZD), lambda i: (0, 0)),
            pl.BlockSpec((_ZD, _H), lambda i: (0, 0)),
            pl.BlockSpec((1, _H), lambda i: (0, 0)),
            pl.BlockSpec((_H, _OD), lambda i: (0, 0)),
            pl.BlockSpec((1, _OD), lambda i: (0, 0)),
        ],
        out_specs=[
            pl.BlockSpec((_LT, _B, _OD), lambda i: (i, 0, 0)),
            pl.BlockSpec((_LT, _B, _ZD), lambda i: (i, 0, 0)),
        ],
        out_shape=[
            jax.ShapeDtypeStruct((_L, _B, _OD), jnp.float32),
            jax.ShapeDtypeStruct((_L, _B, _ZD), jnp.float32),
        ],
        scratch_shapes=[
            pltpu.VMEM((_ZD, _K), jnp.bfloat16),
            pltpu.VMEM((_K, 3 * _ZD), jnp.bfloat16),
            pltpu.VMEM((1, _K), jnp.float32),
            pltpu.VMEM((_ZD, _H), jnp.bfloat16),
            pltpu.VMEM((_H, _OD), jnp.bfloat16),
        ],
    )(xt, embd, W1, b1.reshape(1, _H), W2, b2.reshape(1, _OD))
    out = jnp.transpose(out_t, (1, 0, 2))
    zemb = jnp.transpose(zemb_t, (1, 0, 2))
    return (out, X, zemb)


# speedup vs baseline: 1.0316x; 1.0316x over previous
import jax
import jax.numpy as jnp
from jax.experimental import pallas as pl
from jax.experimental.pallas import tpu as pltpu

_B, _L, _ZD = 16, 196, 256
_K = 1024
_H = 1024
_OD = 768
_LT = 49
_R = _LT * _B
_GRID = _L // _LT


def _vq_body(x_ref, embd_ref, w1_ref, b1_ref, w2_ref, b2_ref,
             out_ref, zemb_ref,
             et_ref, e012_ref, t2h_ref, w1s_ref, w2s_ref):
    @pl.when(pl.program_id(0) == 0)
    def _prep():
        embd = embd_ref[...]
        et = embd.T
        et_ref[...] = et.astype(jnp.bfloat16)
        t2h_ref[...] = 0.5 * jnp.sum(et * et, axis=0, keepdims=True)
        e0 = embd.astype(jnp.bfloat16)
        r1 = embd - e0.astype(jnp.float32)
        e1 = r1.astype(jnp.bfloat16)
        r2 = r1 - e1.astype(jnp.float32)
        e012_ref[...] = jnp.concatenate(
            [e0, e1, r2.astype(jnp.bfloat16)], axis=1)
        w1s_ref[...] = w1_ref[...].astype(jnp.bfloat16)
        w2s_ref[...] = w2_ref[...].astype(jnp.bfloat16)

    dims = (((1,), (0,)), ((), ()))
    x = x_ref[...].reshape(_R, _ZD)
    cross = jax.lax.dot_general(
        x.astype(jnp.bfloat16), et_ref[...], dims,
        preferred_element_type=jnp.float32)
    score = t2h_ref[...] - cross
    idx = jnp.argmin(score, axis=1)
    iota = jax.lax.broadcasted_iota(jnp.int32, (_R, _K), 1)
    onehot = (iota == idx[:, None]).astype(jnp.bfloat16)
    g = jax.lax.dot_general(onehot, e012_ref[...], dims,
                            preferred_element_type=jnp.float32)
    z_emb = g[:, :_ZD] + g[:, _ZD:2 * _ZD] + g[:, 2 * _ZD:]
    zemb_ref[...] = z_emb.reshape(_LT, _B, _ZD)
    h = jax.lax.dot_general(
        z_emb.astype(jnp.bfloat16), w1s_ref[...], dims,
        preferred_element_type=jnp.float32) + b1_ref[...]
    h = jnp.maximum(h, 0.1 * h)
    out = jax.lax.dot_general(
        h.astype(jnp.bfloat16), w2s_ref[...], dims,
        preferred_element_type=jnp.float32) + b2_ref[...]
    out_ref[...] = out.reshape(_LT, _B, _OD)


@jax.jit
def kernel(X, embd, W1, b1, W2, b2):
    xt = jnp.transpose(X, (1, 0, 2))
    out_t, zemb_t = pl.pallas_call(
        _vq_body,
        grid=(_GRID,),
        in_specs=[
            pl.BlockSpec((_LT, _B, _ZD), lambda i: (i, 0, 0)),
            pl.BlockSpec((_K, _ZD), lambda i: (0, 0)),
            pl.BlockSpec((_ZD, _H), lambda i: (0, 0)),
            pl.BlockSpec((1, _H), lambda i: (0, 0)),
            pl.BlockSpec((_H, _OD), lambda i: (0, 0)),
            pl.BlockSpec((1, _OD), lambda i: (0, 0)),
        ],
        out_specs=[
            pl.BlockSpec((_LT, _B, _OD), lambda i: (i, 0, 0)),
            pl.BlockSpec((_LT, _B, _ZD), lambda i: (i, 0, 0)),
        ],
        out_shape=[
            jax.ShapeDtypeStruct((_L, _B, _OD), jnp.float32),
            jax.ShapeDtypeStruct((_L, _B, _ZD), jnp.float32),
        ],
        scratch_shapes=[
            pltpu.VMEM((_ZD, _K), jnp.bfloat16),
            pltpu.VMEM((_K, 3 * _ZD), jnp.bfloat16),
            pltpu.VMEM((1, _K), jnp.float32),
            pltpu.VMEM((_ZD, _H), jnp.bfloat16),
            pltpu.VMEM((_H, _OD), jnp.bfloat16),
        ],
    )(xt, embd, W1, b1.reshape(1, _H), W2, b2.reshape(1, _OD))
    out = jnp.transpose(out_t, (1, 0, 2))
    zemb = jnp.transpose(zemb_t, (1, 0, 2))
    return (out, X, zemb)
